# baseline (speedup 1.0000x reference)
import functools

import jax
import jax.numpy as jnp
from jax.experimental import pallas as pl
from jax.experimental.pallas import tpu as pltpu

TEMP_INV = 1.0 / 0.07
HARD_K = 10
R = 128
W = 384
NCLS = 128


def _rotl(x, d):
    return (x << jnp.uint32(d)) | (x >> jnp.uint32(32 - d))


def _threefry_uniform(p):
    k1 = jnp.uint32(0)
    k2 = jnp.uint32(42)
    kx = k1 ^ k2 ^ jnp.uint32(0x1BD11BDA)
    ks = (k1, k2, kx)
    rot0 = (13, 15, 26, 6)
    rot1 = (17, 29, 16, 24)

    x0 = jnp.zeros_like(p, dtype=jnp.uint32)
    x1 = p.astype(jnp.uint32) + ks[1]
    for rots, a, b, inc in (
        (rot0, 1, 2, 1),
        (rot1, 2, 0, 2),
        (rot0, 0, 1, 3),
        (rot1, 1, 2, 4),
        (rot0, 2, 0, 5),
    ):
        for r in rots:
            x0 = x0 + x1
            x1 = _rotl(x1, r)
            x1 = x0 ^ x1
        x0 = x0 + ks[a]
        x1 = x1 + ks[b] + jnp.uint32(inc)
    bits = x0 ^ x1
    fbits = (bits >> jnp.uint32(9)) | jnp.uint32(0x3F800000)
    return jax.lax.bitcast_convert_type(fbits, jnp.float32) - 1.0


def _pos_select(orig_a, orig_b, same, sim, B):
    u = _threefry_uniform(orig_a * B + orig_b)
    u = jnp.where(same, u, -1.0)
    um = jnp.max(u, axis=1, keepdims=True)
    pos = jnp.sum(jnp.where(u == um, sim, 0.0), axis=1, keepdims=True)
    return um, pos


def _rank_body(keyr_ref, keyc_ref, pos_ref):
    kr = keyr_ref[:, :]
    kc = keyc_ref[:, :]
    pos_ref[:, :] = jnp.sum((kc < kr).astype(jnp.float32), axis=1,
                            keepdims=True)


def _body(pos_ref, labrow_ref, labcol_ref, feats_ref, tot_ref, cnt_ref,
          z_ref, permr_ref, permc_ref, labsc_ref, cumer_ref, cumec_ref,
          nmax_ref, um_ref, plog_ref, *, B, D, G):
    i = pl.program_id(0)
    r0 = i * R

    @pl.when(i == 0)
    def _init():
        tot_ref[:, :] = jnp.zeros((1, 1), jnp.float32)
        cnt_ref[:, :] = jnp.zeros((1, 1), jnp.float32)

        lab_c = labcol_ref[:, :].astype(jnp.float32)
        lab_r = labrow_ref[:, :].astype(jnp.float32)
        cls_r = jax.lax.broadcasted_iota(jnp.int32, (NCLS, 1), 0).astype(jnp.float32)
        cls_c = jax.lax.broadcasted_iota(jnp.int32, (1, NCLS), 1).astype(jnp.float32)
        cnt_col = jnp.sum((lab_c == cls_r).astype(jnp.float32), axis=1,
                          keepdims=True)
        cnt_row = jnp.sum((lab_r == cls_c).astype(jnp.float32), axis=0,
                          keepdims=True)
        io0 = jax.lax.broadcasted_iota(jnp.int32, (NCLS, NCLS), 0).astype(jnp.float32)
        io1 = jax.lax.broadcasted_iota(jnp.int32, (NCLS, NCLS), 1).astype(jnp.float32)
        lt = (io1 <= io0).astype(jnp.float32)
        mt = (io0 <= io1).astype(jnp.float32)
        hi_c = jnp.floor(cnt_col * (1.0 / 64.0))
        lo_c = cnt_col - 64.0 * hi_c
        cumec_ref[:, :] = jnp.floor(
            64.0 * jax.lax.dot_general(
                lt, hi_c, (((1,), (0,)), ((), ())),
                preferred_element_type=jnp.float32) +
            jax.lax.dot_general(
                lt, lo_c, (((1,), (0,)), ((), ())),
                preferred_element_type=jnp.float32) + 0.5)
        hi_r = jnp.floor(cnt_row * (1.0 / 64.0))
        lo_r = cnt_row - 64.0 * hi_r
        cumer_ref[:, :] = jnp.floor(
            64.0 * jax.lax.dot_general(
                hi_r, mt, (((1,), (0,)), ((), ())),
                preferred_element_type=jnp.float32) +
            jax.lax.dot_general(
                lo_r, mt, (((1,), (0,)), ((), ())),
                preferred_element_type=jnp.float32) + 0.5)

        b_c = jax.lax.broadcasted_iota(jnp.int32, (1, B), 1).astype(jnp.float32)
        labsc_ref[:, :] = jnp.sum(
            (b_c >= cumec_ref[:, :]).astype(jnp.float32), axis=0,
            keepdims=True)

        nmax_ref[0, 0] = jnp.max(cnt_col).astype(jnp.int32)

        z_ref[:, :] = jnp.zeros((B, D), jnp.float32)
        permr_ref[:, :] = jnp.zeros((B, 1), jnp.float32)
        permc_ref[:, :] = jnp.zeros((1, B), jnp.float32)
        a_c = jax.lax.broadcasted_iota(jnp.int32, (1, B), 1).astype(jnp.float32)

        def _chunk(k, _):
            sl = pl.ds(k * R, R)
            posk = pos_ref[sl, :]
            oht = (posk == a_c).astype(jnp.float32)
            fk = feats_ref[sl, :]
            nk = jnp.sqrt(jnp.sum(fk * fk, axis=1, keepdims=True))
            zk = fk / jnp.maximum(nk, 1e-12)
            z_ref[:, :] += jax.lax.dot_general(
                oht, zk, (((0,), (0,)), ((), ())),
                preferred_element_type=jnp.float32)
            ic = (k * R + jax.lax.broadcasted_iota(jnp.int32, (R, 1), 0)).astype(jnp.float32)
            ic_hi = jnp.floor(ic * (1.0 / 256.0))
            ic_lo = ic - 256.0 * ic_hi
            permr_ref[:, :] += (
                256.0 * jax.lax.dot_general(
                    oht, ic_hi, (((0,), (0,)), ((), ())),
                    preferred_element_type=jnp.float32) +
                jax.lax.dot_general(
                    oht, ic_lo, (((0,), (0,)), ((), ())),
                    preferred_element_type=jnp.float32))
            permc_ref[:, :] += jnp.sum(oht * ic, axis=0, keepdims=True)
            return 0

        jax.lax.fori_loop(0, G, _chunk, 0)

    zr = z_ref[pl.ds(r0, R), :]
    z = z_ref[:, :]
    sim = jax.lax.dot_general(
        zr, z, (((1,), (1,)), ((), ())),
        preferred_element_type=jnp.float32) * TEMP_INV

    a_row = r0 + jax.lax.broadcasted_iota(jnp.int32, (R, 1), 0)
    lab_row_s = jnp.sum(
        (a_row.astype(jnp.float32) >= cumer_ref[:, :]).astype(jnp.float32),
        axis=1, keepdims=True)
    labsc = labsc_ref[:, :]
    eq = lab_row_s == labsc
    diff = ~eq

    neg = jnp.where(diff, sim, -1e30)
    m1 = jnp.max(neg, axis=1, keepdims=True)
    s = jnp.exp(m1)
    m = m1
    for _ in range(HARD_K - 1):
        m = jnp.max(jnp.where(neg < m, neg, -1e30), axis=1, keepdims=True)
        s = s + jnp.exp(m)

    orig_a = (permr_ref[pl.ds(r0, R), :] + 0.5).astype(jnp.int32)
    ok = nmax_ref[0, 0] <= (W - R) // 2 + 1

    @pl.when(ok)
    def _fast():
        ws = jnp.clip((r0 - (W - R) // 2) // 128, 0, (B - W) // 128) * 128
        zw = z_ref[pl.ds(ws, W), :]
        sim_w = jax.lax.dot_general(
            zr, zw, (((1,), (1,)), ((), ())),
            preferred_element_type=jnp.float32) * TEMP_INV
        bw = ws + jax.lax.broadcasted_iota(jnp.int32, (1, W), 1)
        labw = jnp.sum(
            (bw.astype(jnp.float32) >= cumec_ref[:, :]).astype(jnp.float32),
            axis=0, keepdims=True)
        same_w = (lab_row_s == labw) & (bw != a_row)
        orig_b = (permc_ref[0:1, pl.ds(ws, W)] + 0.5).astype(jnp.int32)
        um, plog = _pos_select(orig_a, orig_b, same_w, sim_w, B)
        um_ref[:, :] = um
        plog_ref[:, :] = plog

    @pl.when(jnp.logical_not(ok))
    def _slow():
        b_full = jax.lax.broadcasted_iota(jnp.int32, (1, B), 1)
        same_f = eq & (b_full != a_row)
        orig_b = (permc_ref[:, :] + 0.5).astype(jnp.int32)
        um, plog = _pos_select(orig_a, orig_b, same_f, sim, B)
        um_ref[:, :] = um
        plog_ref[:, :] = plog

    um = um_ref[:, :]
    pos_logit = plog_ref[:, :]

    num = jnp.exp(pos_logit)
    den = num + s
    loss = -jnp.log(jnp.clip(num / jnp.clip(den, 1e-8, None), 1e-8, None))

    valid = (um >= 0.0) & (m1 > -1e29)
    tot_ref[:, :] += jnp.sum(jnp.where(valid, loss, 0.0), axis=0,
                             keepdims=True)
    cnt_ref[:, :] += jnp.sum(valid.astype(jnp.float32), axis=0,
                             keepdims=True)


def kernel(feats, labels):
    B, D = feats.shape
    G = B // R
    labels_i = labels.astype(jnp.int32)
    key = (labels_i * B + jnp.arange(B, dtype=jnp.int32)).astype(jnp.float32)
    key_r = key.reshape(B, 1)
    key_c = key.reshape(1, B)
    lab_row = labels_i.reshape(B, 1)
    lab_col = labels_i.reshape(1, B)

    pos = pl.pallas_call(
        _rank_body,
        grid=(G,),
        in_specs=[
            pl.BlockSpec((R, 1), lambda i: (i, 0)),
            pl.BlockSpec((1, B), lambda i: (0, 0)),
        ],
        out_specs=pl.BlockSpec((R, 1), lambda i: (i, 0)),
        out_shape=jax.ShapeDtypeStruct((B, 1), jnp.float32),
    )(key_r, key_c)

    tot, cnt = pl.pallas_call(
        functools.partial(_body, B=B, D=D, G=G),
        grid=(G,),
        in_specs=[
            pl.BlockSpec((B, 1), lambda i: (0, 0)),
            pl.BlockSpec((B, 1), lambda i: (0, 0)),
            pl.BlockSpec((1, B), lambda i: (0, 0)),
            pl.BlockSpec((B, D), lambda i: (0, 0)),
        ],
        out_specs=[
            pl.BlockSpec((1, 1), lambda i: (0, 0)),
            pl.BlockSpec((1, 1), lambda i: (0, 0)),
        ],
        out_shape=[
            jax.ShapeDtypeStruct((1, 1), jnp.float32),
            jax.ShapeDtypeStruct((1, 1), jnp.float32),
        ],
        scratch_shapes=[
            pltpu.VMEM((B, D), jnp.float32),
            pltpu.VMEM((B, 1), jnp.float32),
            pltpu.VMEM((1, B), jnp.float32),
            pltpu.VMEM((1, B), jnp.float32),
            pltpu.VMEM((1, NCLS), jnp.float32),
            pltpu.VMEM((NCLS, 1), jnp.float32),
            pltpu.SMEM((1, 1), jnp.int32),
            pltpu.VMEM((R, 1), jnp.float32),
            pltpu.VMEM((R, 1), jnp.float32),
        ],
    )(pos, lab_row, lab_col, feats)

    total = tot[0, 0]
    n_valid = cnt[0, 0]
    return jnp.where(n_valid > 0, total / jnp.maximum(n_valid, 1.0),
                     jnp.zeros(()))


# speedup vs baseline: 1.0890x; 1.0890x over previous
import functools

import jax
import jax.numpy as jnp
from jax.experimental import pallas as pl
from jax.experimental.pallas import tpu as pltpu

TEMP_INV = 1.0 / 0.07
HARD_K = 10
R = 256
W = 512
NCLS = 128


def _rotl(x, d):
    return (x << jnp.uint32(d)) | (x >> jnp.uint32(32 - d))


def _threefry_uniform(p):
    k1 = jnp.uint32(0)
    k2 = jnp.uint32(42)
    kx = k1 ^ k2 ^ jnp.uint32(0x1BD11BDA)
    ks = (k1, k2, kx)
    rot0 = (13, 15, 26, 6)
    rot1 = (17, 29, 16, 24)

    x0 = jnp.zeros_like(p, dtype=jnp.uint32)
    x1 = p.astype(jnp.uint32) + ks[1]
    for rots, a, b, inc in (
        (rot0, 1, 2, 1),
        (rot1, 2, 0, 2),
        (rot0, 0, 1, 3),
        (rot1, 1, 2, 4),
        (rot0, 2, 0, 5),
    ):
        for r in rots:
            x0 = x0 + x1
            x1 = _rotl(x1, r)
            x1 = x0 ^ x1
        x0 = x0 + ks[a]
        x1 = x1 + ks[b] + jnp.uint32(inc)
    bits = x0 ^ x1
    fbits = (bits >> jnp.uint32(9)) | jnp.uint32(0x3F800000)
    return jax.lax.bitcast_convert_type(fbits, jnp.float32) - 1.0


def _pos_select(orig_a, orig_b, same, sim, B):
    u = _threefry_uniform(orig_a * B + orig_b)
    u = jnp.where(same, u, -1.0)
    um = jnp.max(u, axis=1, keepdims=True)
    pos = jnp.sum(jnp.where(u == um, sim, 0.0), axis=1, keepdims=True)
    return um, pos


def _body(labrow_ref, labcol_ref, feats_ref, tot_ref, cnt_ref,
          z_ref, permr_ref, permc_ref, labsc_ref, ohc_ref, cumer_ref,
          cumec_ref, nmax_ref, um_ref, plog_ref, *, B, D, G):
    i = pl.program_id(0)
    r0 = i * R

    @pl.when(i == 0)
    def _init():
        tot_ref[:, :] = jnp.zeros((1, 1), jnp.float32)
        cnt_ref[:, :] = jnp.zeros((1, 1), jnp.float32)

        lab_c = labcol_ref[:, :].astype(jnp.float32)
        lab_r = labrow_ref[:, :].astype(jnp.float32)
        cls_r = jax.lax.broadcasted_iota(jnp.int32, (NCLS, 1), 0).astype(jnp.float32)
        cls_c = jax.lax.broadcasted_iota(jnp.int32, (1, NCLS), 1).astype(jnp.float32)
        cnt_col = jnp.sum((lab_c == cls_r).astype(jnp.float32), axis=1,
                          keepdims=True)
        cnt_row = jnp.sum((lab_r == cls_c).astype(jnp.float32), axis=0,
                          keepdims=True)
        io0 = jax.lax.broadcasted_iota(jnp.int32, (NCLS, NCLS), 0).astype(jnp.float32)
        io1 = jax.lax.broadcasted_iota(jnp.int32, (NCLS, NCLS), 1).astype(jnp.float32)
        lt = (io1 <= io0).astype(jnp.float32)
        mt = (io0 <= io1).astype(jnp.float32)
        hi_c = jnp.floor(cnt_col * (1.0 / 64.0))
        lo_c = cnt_col - 64.0 * hi_c
        cumec_ref[:, :] = jnp.floor(
            64.0 * jax.lax.dot_general(
                lt, hi_c, (((1,), (0,)), ((), ())),
                preferred_element_type=jnp.float32) +
            jax.lax.dot_general(
                lt, lo_c, (((1,), (0,)), ((), ())),
                preferred_element_type=jnp.float32) + 0.5)
        hi_r = jnp.floor(cnt_row * (1.0 / 64.0))
        lo_r = cnt_row - 64.0 * hi_r
        cumer_ref[:, :] = jnp.floor(
            64.0 * jax.lax.dot_general(
                hi_r, mt, (((1,), (0,)), ((), ())),
                preferred_element_type=jnp.float32) +
            jax.lax.dot_general(
                lo_r, mt, (((1,), (0,)), ((), ())),
                preferred_element_type=jnp.float32) + 0.5)

        b_c = jax.lax.broadcasted_iota(jnp.int32, (1, B), 1).astype(jnp.float32)
        labsc_ref[:, :] = jnp.sum(
            (b_c >= cumec_ref[:, :]).astype(jnp.float32), axis=0,
            keepdims=True)
        ohc_ref[:, :] = (labsc_ref[:, :] == cls_r).astype(jnp.float32)

        nmax_ref[0, 0] = jnp.max(cnt_col).astype(jnp.int32)

        z_ref[:, :] = jnp.zeros((B, D), jnp.float32)
        permr_ref[:, :] = jnp.zeros((B, 1), jnp.float32)
        permc_ref[:, :] = jnp.zeros((1, B), jnp.float32)
        a_c = jax.lax.broadcasted_iota(jnp.int32, (1, B), 1).astype(jnp.float32)
        key_c = lab_c * B + a_c

        def _chunk(k, _):
            sl = pl.ds(k * R, R)
            ick = (k * R + jax.lax.broadcasted_iota(
                jnp.int32, (R, 1), 0)).astype(jnp.float32)
            key_k = labrow_ref[sl, :].astype(jnp.float32) * B + ick
            posk = jnp.sum((key_c < key_k).astype(jnp.float32), axis=1,
                           keepdims=True)
            oht = (posk == a_c).astype(jnp.float32)
            fk = feats_ref[sl, :]
            nk = jnp.sqrt(jnp.sum(fk * fk, axis=1, keepdims=True))
            zk = fk / jnp.maximum(nk, 1e-12)
            z_ref[:, :] += jax.lax.dot_general(
                oht, zk, (((0,), (0,)), ((), ())),
                preferred_element_type=jnp.float32)
            ic = ick
            ic_hi = jnp.floor(ic * (1.0 / 256.0))
            ic_lo = ic - 256.0 * ic_hi
            permr_ref[:, :] += (
                256.0 * jax.lax.dot_general(
                    oht, ic_hi, (((0,), (0,)), ((), ())),
                    preferred_element_type=jnp.float32) +
                jax.lax.dot_general(
                    oht, ic_lo, (((0,), (0,)), ((), ())),
                    preferred_element_type=jnp.float32))
            permc_ref[:, :] += jnp.sum(oht * ic, axis=0, keepdims=True)
            return 0

        jax.lax.fori_loop(0, G, _chunk, 0)

    zr = z_ref[pl.ds(r0, R), :]
    z = z_ref[:, :]
    sim = jax.lax.dot_general(
        zr, z, (((1,), (1,)), ((), ())),
        preferred_element_type=jnp.float32) * TEMP_INV

    a_row = r0 + jax.lax.broadcasted_iota(jnp.int32, (R, 1), 0)
    lab_row_s = jnp.sum(
        (a_row.astype(jnp.float32) >= cumer_ref[:, :]).astype(jnp.float32),
        axis=1, keepdims=True)
    cls_c2 = jax.lax.broadcasted_iota(jnp.int32, (1, NCLS), 1).astype(
        jnp.float32)
    oh_row = (lab_row_s == cls_c2).astype(jnp.float32)
    eqf = jax.lax.dot_general(
        oh_row, ohc_ref[:, :], (((1,), (0,)), ((), ())),
        preferred_element_type=jnp.float32)

    neg = sim - eqf * 1e30
    m1 = jnp.max(neg, axis=1, keepdims=True)
    s = jnp.exp(m1)
    m = m1
    for _ in range(HARD_K - 1):
        m = jnp.max(jnp.where(neg < m, neg, -1e30), axis=1, keepdims=True)
        s = s + jnp.exp(m)

    orig_a = (permr_ref[pl.ds(r0, R), :] + 0.5).astype(jnp.int32)
    ok = nmax_ref[0, 0] <= (W - R) // 2 + 1

    @pl.when(ok)
    def _fast():
        ws = jnp.clip((r0 - (W - R) // 2) // 128, 0, (B - W) // 128) * 128
        zw = z_ref[pl.ds(ws, W), :]
        sim_w = jax.lax.dot_general(
            zr, zw, (((1,), (1,)), ((), ())),
            preferred_element_type=jnp.float32) * TEMP_INV
        bw = ws + jax.lax.broadcasted_iota(jnp.int32, (1, W), 1)
        labw = jnp.sum(
            (bw.astype(jnp.float32) >= cumec_ref[:, :]).astype(jnp.float32),
            axis=0, keepdims=True)
        same_w = (lab_row_s == labw) & (bw != a_row)
        orig_b = (permc_ref[0:1, pl.ds(ws, W)] + 0.5).astype(jnp.int32)
        um, plog = _pos_select(orig_a, orig_b, same_w, sim_w, B)
        um_ref[:, :] = um
        plog_ref[:, :] = plog

    @pl.when(jnp.logical_not(ok))
    def _slow():
        b_full = jax.lax.broadcasted_iota(jnp.int32, (1, B), 1)
        same_f = (eqf > 0.5) & (b_full != a_row)
        orig_b = (permc_ref[:, :] + 0.5).astype(jnp.int32)
        um, plog = _pos_select(orig_a, orig_b, same_f, sim, B)
        um_ref[:, :] = um
        plog_ref[:, :] = plog

    um = um_ref[:, :]
    pos_logit = plog_ref[:, :]

    num = jnp.exp(pos_logit)
    den = num + s
    loss = -jnp.log(jnp.clip(num / jnp.clip(den, 1e-8, None), 1e-8, None))

    valid = (um >= 0.0) & (m1 > -1e29)
    tot_ref[:, :] += jnp.sum(jnp.where(valid, loss, 0.0), axis=0,
                             keepdims=True)
    cnt_ref[:, :] += jnp.sum(valid.astype(jnp.float32), axis=0,
                             keepdims=True)


def kernel(feats, labels):
    B, D = feats.shape
    G = B // R
    labels_i = labels.astype(jnp.int32)
    lab_row = labels_i.reshape(B, 1)
    lab_col = labels_i.reshape(1, B)

    tot, cnt = pl.pallas_call(
        functools.partial(_body, B=B, D=D, G=G),
        grid=(G,),
        in_specs=[
            pl.BlockSpec((B, 1), lambda i: (0, 0)),
            pl.BlockSpec((1, B), lambda i: (0, 0)),
            pl.BlockSpec((B, D), lambda i: (0, 0)),
        ],
        out_specs=[
            pl.BlockSpec((1, 1), lambda i: (0, 0)),
            pl.BlockSpec((1, 1), lambda i: (0, 0)),
        ],
        out_shape=[
            jax.ShapeDtypeStruct((1, 1), jnp.float32),
            jax.ShapeDtypeStruct((1, 1), jnp.float32),
        ],
        scratch_shapes=[
            pltpu.VMEM((B, D), jnp.float32),
            pltpu.VMEM((B, 1), jnp.float32),
            pltpu.VMEM((1, B), jnp.float32),
            pltpu.VMEM((1, B), jnp.float32),
            pltpu.VMEM((NCLS, B), jnp.float32),
            pltpu.VMEM((1, NCLS), jnp.float32),
            pltpu.VMEM((NCLS, 1), jnp.float32),
            pltpu.SMEM((1, 1), jnp.int32),
            pltpu.VMEM((R, 1), jnp.float32),
            pltpu.VMEM((R, 1), jnp.float32),
        ],
    )(lab_row, lab_col, feats)

    total = tot[0, 0]
    n_valid = cnt[0, 0]
    return jnp.where(n_valid > 0, total / jnp.maximum(n_valid, 1.0),
                     jnp.zeros(()))
